# baseline (speedup 1.0000x reference)
import functools

import numpy as np
import jax
import jax.numpy as jnp
from jax import lax
from jax.experimental import pallas as pl
from jax.experimental.pallas import tpu as pltpu
from jax.experimental.pallas import tpu_sc as plsc

_B, _L, _D = 4, 4096, 768
_N = _B * _L
_NC, _NS, _LANES = 2, 16, 16
_NW = _NC * _NS
_BPW = _N // _NW
_K = 32
_NCHUNK = _BPW // _K


def _pe_table_np():
    pos = np.arange(_L, dtype=np.float32)[:, None]
    i2 = np.arange(0, _D, 2, dtype=np.float32)
    div = np.power(10000.0, i2 / float(_D))
    enc = np.zeros((_L, _D), dtype=np.float32)
    enc[:, 0::2] = np.sin(pos / div)
    enc[:, 1::2] = np.cos(pos / div)
    return enc


_PE_NP = _pe_table_np()

_mesh = plsc.VectorSubcoreMesh(core_axis_name="c", subcore_axis_name="s")


@functools.partial(
    pl.kernel,
    mesh=_mesh,
    out_type=jax.ShapeDtypeStruct((_N, _D), jnp.float32),
    scratch_types=[
        pltpu.VMEM((_BPW,), jnp.int32),
        pltpu.VMEM((_K, _D), jnp.float32),
        pltpu.VMEM((_K, _D), jnp.float32),
        pltpu.SemaphoreType.DMA,
        pltpu.SemaphoreType.DMA,
    ],
)
def _emb(idx_hbm, table_hbm, pe_hbm, out_hbm, idx_v, tok_v, pe_v, sem_tok, sem_pe):
    wid = lax.axis_index("s") * _NC + lax.axis_index("c")
    base = wid * _BPW
    pbase = base % _L
    pltpu.sync_copy(idx_hbm.at[pl.ds(base, _BPW)], idx_v)
    for c in range(_NCHUNK):
        tok_cp = pltpu.async_copy(
            table_hbm.at[idx_v.at[pl.ds(c * _K, _K)]], tok_v, sem_tok)
        pe_cp = pltpu.async_copy(
            pe_hbm.at[pl.ds(pbase + c * _K, _K)], pe_v, sem_pe)
        tok_cp.wait()
        pe_cp.wait()

        def add_row(r, carry):
            for j in range(_D // _LANES):
                sl = pl.ds(j * _LANES, _LANES)
                tok_v[r, sl] = tok_v[r, sl] + pe_v[r, sl]
            return carry

        lax.fori_loop(0, _K, add_row, 0)
        pltpu.sync_copy(tok_v, out_hbm.at[pl.ds(base + c * _K, _K)])


def kernel(x, table):
    pe = jnp.asarray(_PE_NP)
    out = _emb(x.reshape(_N), table, pe)
    return out.reshape(_B, _L, _D)


# speedup vs baseline: 1.1019x; 1.1019x over previous
import functools

import numpy as np
import jax
import jax.numpy as jnp
from jax import lax
from jax.experimental import pallas as pl
from jax.experimental.pallas import tpu as pltpu
from jax.experimental.pallas import tpu_sc as plsc

_B, _L, _D = 4, 4096, 768
_N = _B * _L
_NC, _NS, _LANES = 2, 16, 16
_NW = _NC * _NS
_PPW = _L // _NW
_K = 32
_NPC = _PPW // _K
_NT = _NPC * _B


def _pe_table_np():
    pos = np.arange(_L, dtype=np.float32)[:, None]
    i2 = np.arange(0, _D, 2, dtype=np.float32)
    div = np.power(10000.0, i2 / float(_D))
    enc = np.zeros((_L, _D), dtype=np.float32)
    enc[:, 0::2] = np.sin(pos / div)
    enc[:, 1::2] = np.cos(pos / div)
    return enc


_PE_NP = _pe_table_np()

_mesh = plsc.VectorSubcoreMesh(core_axis_name="c", subcore_axis_name="s")


@functools.partial(
    pl.kernel,
    mesh=_mesh,
    out_type=jax.ShapeDtypeStruct((_N, _D), jnp.float32),
    scratch_types=[
        pltpu.VMEM((_B * _PPW,), jnp.int32),
        pltpu.VMEM((3, _K, _D), jnp.float32),
        pltpu.VMEM((2, _K, _D), jnp.float32),
        pltpu.SemaphoreType.DMA,
        pltpu.SemaphoreType.DMA,
        pltpu.SemaphoreType.DMA,
        pltpu.SemaphoreType.DMA,
        pltpu.SemaphoreType.DMA,
        pltpu.SemaphoreType.DMA,
        pltpu.SemaphoreType.DMA,
        pltpu.SemaphoreType.DMA,
    ],
)
def _emb(idx_hbm, table_hbm, pe_hbm, out_hbm, idx_v, tok_v, pe_v,
         st0, st1, st2, sp0, sp1, so0, so1, so2):
    sem_tok = (st0, st1, st2)
    sem_pe = (sp0, sp1)
    sem_out = (so0, so1, so2)
    wid = lax.axis_index("s") * _NC + lax.axis_index("c")
    pos0 = wid * _PPW

    for b in range(_B):
        pltpu.sync_copy(idx_hbm.at[pl.ds(b * _L + pos0, _PPW)],
                        idx_v.at[pl.ds(b * _PPW, _PPW)])

    def start_tok(t):
        c, b = t // _B, t % _B
        return pltpu.async_copy(
            table_hbm.at[idx_v.at[pl.ds(b * _PPW + c * _K, _K)]],
            tok_v.at[t % 3], sem_tok[t % 3])

    def start_pe(c):
        return pltpu.async_copy(
            pe_hbm.at[pl.ds(pos0 + c * _K, _K)],
            pe_v.at[c % 2], sem_pe[c % 2])

    def start_out(t):
        c, b = t // _B, t % _B
        return pltpu.async_copy(
            tok_v.at[t % 3],
            out_hbm.at[pl.ds(b * _L + pos0 + c * _K, _K)], sem_out[t % 3])

    pe_cp = {0: start_pe(0)}
    tok_cp = {0: start_tok(0), 1: start_tok(1)}
    out_cp = {}

    for t in range(_NT):
        c, b = t // _B, t % _B
        if t + 2 < _NT:
            if t >= 1:
                out_cp[t - 1].wait()
            tok_cp[t + 2] = start_tok(t + 2)
        if b == 0:
            pe_cp[c].wait()
            if c + 1 < _NPC:
                pe_cp[c + 1] = start_pe(c + 1)
        tok_cp[t].wait()

        def add_row(r, carry):
            for j in range(_D // _LANES):
                sl = pl.ds(j * _LANES, _LANES)
                plsc.addupdate(tok_v.at[t % 3, r, sl], pe_v[c % 2, r, sl])
            return carry

        lax.fori_loop(0, _K, add_row, 0)
        out_cp[t] = start_out(t)

    for t in range(_NT - 3, _NT):
        out_cp[t].wait()


def kernel(x, table):
    pe = jnp.asarray(_PE_NP)
    out = _emb(x.reshape(_N), table, pe)
    return out.reshape(_B, _L, _D)


# speedup vs baseline: 1.5512x; 1.4077x over previous
import functools

import numpy as np
import jax
import jax.numpy as jnp
from jax import lax
from jax.experimental import pallas as pl
from jax.experimental.pallas import tpu as pltpu
from jax.experimental.pallas import tpu_sc as plsc

_B, _L, _D = 4, 4096, 768
_N = _B * _L
_NC, _NS, _LANES = 2, 16, 16
_NW = _NC * _NS
_PPW = _L // _NW
_K = 16
_NPC = _PPW // _K
_NT = _NPC * _B
_RB = 4
_LOOK = 2


def _pe_table_np():
    pos = np.arange(_L, dtype=np.float32)[:, None]
    i2 = np.arange(0, _D, 2, dtype=np.float32)
    div = np.power(10000.0, i2 / float(_D))
    enc = np.zeros((_L, _D), dtype=np.float32)
    enc[:, 0::2] = np.sin(pos / div)
    enc[:, 1::2] = np.cos(pos / div)
    return enc


_PE_NP = _pe_table_np()

_mesh = plsc.VectorSubcoreMesh(core_axis_name="c", subcore_axis_name="s")


@functools.partial(
    pl.kernel,
    mesh=_mesh,
    out_type=jax.ShapeDtypeStruct((_N, _D), jnp.float32),
    scratch_types=[
        pltpu.VMEM((_B * _PPW,), jnp.int32),
        pltpu.VMEM((_RB, _K, _D), jnp.float32),
        pltpu.VMEM((2, _K, _D), jnp.float32),
        pltpu.SemaphoreType.DMA,
        pltpu.SemaphoreType.DMA,
        pltpu.SemaphoreType.DMA,
        pltpu.SemaphoreType.DMA,
        pltpu.SemaphoreType.DMA,
        pltpu.SemaphoreType.DMA,
        pltpu.SemaphoreType.DMA,
        pltpu.SemaphoreType.DMA,
        pltpu.SemaphoreType.DMA,
        pltpu.SemaphoreType.DMA,
    ],
)
def _emb(idx_hbm, table_hbm, pe_hbm, out_hbm, idx_v, tok_v, pe_v,
         st0, st1, st2, st3, sp0, sp1, so0, so1, so2, so3):
    sem_tok = (st0, st1, st2, st3)
    sem_pe = (sp0, sp1)
    sem_out = (so0, so1, so2, so3)
    wid = lax.axis_index("s") * _NC + lax.axis_index("c")
    pos0 = wid * _PPW

    for b in range(_B):
        pltpu.sync_copy(idx_hbm.at[pl.ds(b * _L + pos0, _PPW)],
                        idx_v.at[pl.ds(b * _PPW, _PPW)])

    def start_tok(t):
        c, b = t // _B, t % _B
        return pltpu.async_copy(
            table_hbm.at[idx_v.at[pl.ds(b * _PPW + c * _K, _K)]],
            tok_v.at[t % _RB], sem_tok[t % _RB])

    def start_pe(c):
        return pltpu.async_copy(
            pe_hbm.at[pl.ds(pos0 + c * _K, _K)],
            pe_v.at[c % 2], sem_pe[c % 2])

    def start_out(t):
        c, b = t // _B, t % _B
        return pltpu.async_copy(
            tok_v.at[t % _RB],
            out_hbm.at[pl.ds(b * _L + pos0 + c * _K, _K)], sem_out[t % _RB])

    pe_cp = {0: start_pe(0), 1: start_pe(1)}
    tok_cp = {t: start_tok(t) for t in range(_LOOK)}
    out_cp = {}

    for t in range(_NT):
        c, b = t // _B, t % _B
        if b == 0:
            pe_cp[c].wait()
        tok_cp[t].wait()

        def add_row(r, carry):
            @plsc.parallel_loop(0, _D, _LANES, unroll=8)
            def add_lane(j):
                plsc.addupdate(tok_v.at[t % _RB, r, pl.ds(j, _LANES)],
                               pe_v[c % 2, r, pl.ds(j, _LANES)])
            return carry

        lax.fori_loop(0, _K, add_row, 0)
        out_cp[t] = start_out(t)
        if b == _B - 1 and c + 2 < _NPC:
            pe_cp[c + 2] = start_pe(c + 2)
        if t + _LOOK < _NT:
            if t + _LOOK - _RB >= 0:
                out_cp[t + _LOOK - _RB].wait()
            tok_cp[t + _LOOK] = start_tok(t + _LOOK)

    for t in range(_NT - _RB, _NT):
        out_cp[t].wait()


def kernel(x, table):
    pe = jnp.asarray(_PE_NP)
    out = _emb(x.reshape(_N), table, pe)
    return out.reshape(_B, _L, _D)


# speedup vs baseline: 1.6490x; 1.0631x over previous
import functools

import numpy as np
import jax
import jax.numpy as jnp
from jax import lax
from jax.experimental import pallas as pl
from jax.experimental.pallas import tpu as pltpu
from jax.experimental.pallas import tpu_sc as plsc

_B, _L, _D = 4, 4096, 768
_N = _B * _L
_NC, _NS, _LANES = 2, 16, 16
_NW = _NC * _NS
_PPW = _L // _NW
_K = 16
_NPC = _PPW // _K
_NT = _NPC * _B
_RB = 6
_LOOK = 3


def _pe_table_np():
    pos = np.arange(_L, dtype=np.float32)[:, None]
    i2 = np.arange(0, _D, 2, dtype=np.float32)
    div = np.power(10000.0, i2 / float(_D))
    enc = np.zeros((_L, _D), dtype=np.float32)
    enc[:, 0::2] = np.sin(pos / div)
    enc[:, 1::2] = np.cos(pos / div)
    return enc


_PE_NP = _pe_table_np()

_mesh = plsc.VectorSubcoreMesh(core_axis_name="c", subcore_axis_name="s")


@functools.partial(
    pl.kernel,
    mesh=_mesh,
    out_type=jax.ShapeDtypeStruct((_N, _D), jnp.float32),
    scratch_types=[
        pltpu.VMEM((_B * _PPW,), jnp.int32),
        pltpu.VMEM((_RB, _K, _D), jnp.float32),
        pltpu.VMEM((2, _K, _D), jnp.float32),
        pltpu.SemaphoreType.DMA,
        pltpu.SemaphoreType.DMA,
        pltpu.SemaphoreType.DMA,
        pltpu.SemaphoreType.DMA,
        pltpu.SemaphoreType.DMA,
        pltpu.SemaphoreType.DMA,
        pltpu.SemaphoreType.DMA,
        pltpu.SemaphoreType.DMA,
        pltpu.SemaphoreType.DMA,
        pltpu.SemaphoreType.DMA,
        pltpu.SemaphoreType.DMA,
        pltpu.SemaphoreType.DMA,
        pltpu.SemaphoreType.DMA,
        pltpu.SemaphoreType.DMA,
    ],
)
def _emb(idx_hbm, table_hbm, pe_hbm, out_hbm, idx_v, tok_v, pe_v,
         st0, st1, st2, st3, st4, st5, sp0, sp1,
         so0, so1, so2, so3, so4, so5):
    sem_tok = (st0, st1, st2, st3, st4, st5)
    sem_pe = (sp0, sp1)
    sem_out = (so0, so1, so2, so3, so4, so5)
    wid = lax.axis_index("s") * _NC + lax.axis_index("c")
    pos0 = wid * _PPW

    for b in range(_B):
        pltpu.sync_copy(idx_hbm.at[pl.ds(b * _L + pos0, _PPW)],
                        idx_v.at[pl.ds(b * _PPW, _PPW)])

    def start_tok(t):
        c, b = t // _B, t % _B
        return pltpu.async_copy(
            table_hbm.at[idx_v.at[pl.ds(b * _PPW + c * _K, _K)]],
            tok_v.at[t % _RB], sem_tok[t % _RB])

    def start_pe(c):
        return pltpu.async_copy(
            pe_hbm.at[pl.ds(pos0 + c * _K, _K)],
            pe_v.at[c % 2], sem_pe[c % 2])

    def start_out(t):
        c, b = t // _B, t % _B
        return pltpu.async_copy(
            tok_v.at[t % _RB],
            out_hbm.at[pl.ds(b * _L + pos0 + c * _K, _K)], sem_out[t % _RB])

    pe_cp = {0: start_pe(0), 1: start_pe(1)}
    tok_cp = {t: start_tok(t) for t in range(_LOOK)}
    out_cp = {}

    for t in range(_NT):
        c, b = t // _B, t % _B
        if b == 0:
            pe_cp[c].wait()
        tok_cp[t].wait()

        def add_row(r, carry):
            @plsc.parallel_loop(0, _D, _LANES, unroll=8)
            def add_lane(j):
                plsc.addupdate(tok_v.at[t % _RB, r, pl.ds(j, _LANES)],
                               pe_v[c % 2, r, pl.ds(j, _LANES)])
            return carry

        lax.fori_loop(0, _K, add_row, 0)
        out_cp[t] = start_out(t)
        if b == _B - 1 and c + 2 < _NPC:
            pe_cp[c + 2] = start_pe(c + 2)
        if t + _LOOK < _NT:
            if t + _LOOK - _RB >= 0:
                out_cp[t + _LOOK - _RB].wait()
            tok_cp[t + _LOOK] = start_tok(t + _LOOK)

    for t in range(_NT - _RB, _NT):
        out_cp[t].wait()


def kernel(x, table):
    pe = jnp.asarray(_PE_NP)
    out = _emb(x.reshape(_N), table, pe)
    return out.reshape(_B, _L, _D)


# speedup vs baseline: 1.6834x; 1.0208x over previous
import functools

import numpy as np
import jax
import jax.numpy as jnp
from jax import lax
from jax.experimental import pallas as pl
from jax.experimental.pallas import tpu as pltpu
from jax.experimental.pallas import tpu_sc as plsc

_B, _L, _D = 4, 4096, 768
_N = _B * _L
_NC, _NS, _LANES = 2, 16, 16
_NW = _NC * _NS
_PPW = _L // _NW
_K = 16
_NPC = _PPW // _K
_NT = _NPC * _B
_RB = 6
_LOOK = 3


def _pe_table_np():
    pos = np.arange(_L, dtype=np.float32)[:, None]
    i2 = np.arange(0, _D, 2, dtype=np.float32)
    div = np.power(10000.0, i2 / float(_D))
    enc = np.zeros((_L, _D), dtype=np.float32)
    enc[:, 0::2] = np.sin(pos / div)
    enc[:, 1::2] = np.cos(pos / div)
    return enc


_PE_NP = _pe_table_np()

_mesh = plsc.VectorSubcoreMesh(core_axis_name="c", subcore_axis_name="s")


@functools.partial(
    pl.kernel,
    mesh=_mesh,
    out_type=jax.ShapeDtypeStruct((_B, _L, _D), jnp.float32),
    scratch_types=[
        pltpu.VMEM((_B * _PPW,), jnp.int32),
        pltpu.VMEM((_RB, _K, _D), jnp.float32),
        pltpu.VMEM((2, _K, _D), jnp.float32),
        pltpu.SemaphoreType.DMA,
        pltpu.SemaphoreType.DMA,
        pltpu.SemaphoreType.DMA,
        pltpu.SemaphoreType.DMA,
        pltpu.SemaphoreType.DMA,
        pltpu.SemaphoreType.DMA,
        pltpu.SemaphoreType.DMA,
        pltpu.SemaphoreType.DMA,
        pltpu.SemaphoreType.DMA,
        pltpu.SemaphoreType.DMA,
        pltpu.SemaphoreType.DMA,
        pltpu.SemaphoreType.DMA,
        pltpu.SemaphoreType.DMA,
        pltpu.SemaphoreType.DMA,
    ],
)
def _emb(idx_hbm, table_hbm, pe_hbm, out_hbm, idx_v, tok_v, pe_v,
         st0, st1, st2, st3, st4, st5, sp0, sp1,
         so0, so1, so2, so3, so4, so5):
    sem_tok = (st0, st1, st2, st3, st4, st5)
    sem_pe = (sp0, sp1)
    sem_out = (so0, so1, so2, so3, so4, so5)
    wid = lax.axis_index("s") * _NC + lax.axis_index("c")
    pos0 = wid * _PPW

    for b in range(_B):
        pltpu.sync_copy(idx_hbm.at[b, pl.ds(pos0, _PPW)],
                        idx_v.at[pl.ds(b * _PPW, _PPW)])

    def start_tok(t):
        c, b = t // _B, t % _B
        return pltpu.async_copy(
            table_hbm.at[idx_v.at[pl.ds(b * _PPW + c * _K, _K)]],
            tok_v.at[t % _RB], sem_tok[t % _RB])

    def start_pe(c):
        return pltpu.async_copy(
            pe_hbm.at[pl.ds(pos0 + c * _K, _K)],
            pe_v.at[c % 2], sem_pe[c % 2])

    def start_out(t):
        c, b = t // _B, t % _B
        return pltpu.async_copy(
            tok_v.at[t % _RB],
            out_hbm.at[b, pl.ds(pos0 + c * _K, _K)], sem_out[t % _RB])

    pe_cp = {0: start_pe(0), 1: start_pe(1)}
    tok_cp = {t: start_tok(t) for t in range(_LOOK)}
    out_cp = {}

    for t in range(_NT):
        c, b = t // _B, t % _B
        if b == 0:
            pe_cp[c].wait()
        tok_cp[t].wait()

        def add_row(r, carry):
            @plsc.parallel_loop(0, _D, _LANES, unroll=8)
            def add_lane(j):
                plsc.addupdate(tok_v.at[t % _RB, r, pl.ds(j, _LANES)],
                               pe_v[c % 2, r, pl.ds(j, _LANES)])
            return carry

        lax.fori_loop(0, _K, add_row, 0)
        out_cp[t] = start_out(t)
        if b == _B - 1 and c + 2 < _NPC:
            pe_cp[c + 2] = start_pe(c + 2)
        if t + _LOOK < _NT:
            if t + _LOOK - _RB >= 0:
                out_cp[t + _LOOK - _RB].wait()
            tok_cp[t + _LOOK] = start_tok(t + _LOOK)

    for t in range(_NT - _RB, _NT):
        out_cp[t].wait()


def kernel(x, table):
    pe = jnp.asarray(_PE_NP)
    return _emb(x, table, pe)


# speedup vs baseline: 1.6912x; 1.0047x over previous
import functools

import numpy as np
import jax
import jax.numpy as jnp
from jax import lax
from jax.experimental import pallas as pl
from jax.experimental.pallas import tpu as pltpu
from jax.experimental.pallas import tpu_sc as plsc

_B, _L, _D = 4, 4096, 768
_N = _B * _L
_NC, _NS, _LANES = 2, 16, 16
_NW = _NC * _NS
_PPW = _L // _NW
_K = 16
_NPC = _PPW // _K
_NT = _NPC * _B
_RB = 6
_LOOK = 3


def _pe_table_np():
    pos = np.arange(_L, dtype=np.float32)[:, None]
    i2 = np.arange(0, _D, 2, dtype=np.float32)
    div = np.power(10000.0, i2 / float(_D))
    enc = np.zeros((_L, _D), dtype=np.float32)
    enc[:, 0::2] = np.sin(pos / div)
    enc[:, 1::2] = np.cos(pos / div)
    return enc


_PE_NP = _pe_table_np()
_PE_DEV = None


def _pe_device():
    global _PE_DEV
    if _PE_DEV is None:
        _PE_DEV = jnp.asarray(_PE_NP)
    return _PE_DEV

_mesh = plsc.VectorSubcoreMesh(core_axis_name="c", subcore_axis_name="s")


@functools.partial(
    pl.kernel,
    mesh=_mesh,
    out_type=jax.ShapeDtypeStruct((_B, _L, _D), jnp.float32),
    scratch_types=[
        pltpu.VMEM((_B * _PPW,), jnp.int32),
        pltpu.VMEM((_RB, _K, _D), jnp.float32),
        pltpu.VMEM((2, _K, _D), jnp.float32),
        pltpu.SemaphoreType.DMA,
        pltpu.SemaphoreType.DMA,
        pltpu.SemaphoreType.DMA,
        pltpu.SemaphoreType.DMA,
        pltpu.SemaphoreType.DMA,
        pltpu.SemaphoreType.DMA,
        pltpu.SemaphoreType.DMA,
        pltpu.SemaphoreType.DMA,
        pltpu.SemaphoreType.DMA,
        pltpu.SemaphoreType.DMA,
        pltpu.SemaphoreType.DMA,
        pltpu.SemaphoreType.DMA,
        pltpu.SemaphoreType.DMA,
        pltpu.SemaphoreType.DMA,
    ],
)
def _emb(idx_hbm, table_hbm, pe_hbm, out_hbm, idx_v, tok_v, pe_v,
         st0, st1, st2, st3, st4, st5, sp0, sp1,
         so0, so1, so2, so3, so4, so5):
    sem_tok = (st0, st1, st2, st3, st4, st5)
    sem_pe = (sp0, sp1)
    sem_out = (so0, so1, so2, so3, so4, so5)
    wid = lax.axis_index("s") * _NC + lax.axis_index("c")
    pos0 = wid * _PPW

    for b in range(_B):
        pltpu.sync_copy(idx_hbm.at[b, pl.ds(pos0, _PPW)],
                        idx_v.at[pl.ds(b * _PPW, _PPW)])

    def start_tok(t):
        c, b = t // _B, t % _B
        return pltpu.async_copy(
            table_hbm.at[idx_v.at[pl.ds(b * _PPW + c * _K, _K)]],
            tok_v.at[t % _RB], sem_tok[t % _RB])

    def start_pe(c):
        return pltpu.async_copy(
            pe_hbm.at[pl.ds(pos0 + c * _K, _K)],
            pe_v.at[c % 2], sem_pe[c % 2])

    def start_out(t):
        c, b = t // _B, t % _B
        return pltpu.async_copy(
            tok_v.at[t % _RB],
            out_hbm.at[b, pl.ds(pos0 + c * _K, _K)], sem_out[t % _RB])

    pe_cp = {0: start_pe(0), 1: start_pe(1)}
    tok_cp = {t: start_tok(t) for t in range(_LOOK)}
    out_cp = {}

    for t in range(_NT):
        c, b = t // _B, t % _B
        if b == 0:
            pe_cp[c].wait()
        tok_cp[t].wait()

        def add_row(r, carry):
            @plsc.parallel_loop(0, _D, _LANES, unroll=8)
            def add_lane(j):
                plsc.addupdate(tok_v.at[t % _RB, r, pl.ds(j, _LANES)],
                               pe_v[c % 2, r, pl.ds(j, _LANES)])
            return carry

        lax.fori_loop(0, _K, add_row, 0)
        out_cp[t] = start_out(t)
        if b == _B - 1 and c + 2 < _NPC:
            pe_cp[c + 2] = start_pe(c + 2)
        if t + _LOOK < _NT:
            if t + _LOOK - _RB >= 0:
                out_cp[t + _LOOK - _RB].wait()
            tok_cp[t + _LOOK] = start_tok(t + _LOOK)

    for t in range(_NT - _RB, _NT):
        out_cp[t].wait()


def kernel(x, table):
    return _emb(x, table, _pe_device())
